# Initial kernel scaffold; baseline (speedup 1.0000x reference)
import jax
import jax.numpy as jnp
from jax.experimental import pallas as pl


def kernel(token_ids, embedding_weights):
    raise NotImplementedError("write your pallas kernel here")


# speedup vs baseline: 1.8321x; 1.8321x over previous
import functools

import jax
import jax.numpy as jnp
from jax import lax
from jax.experimental import pallas as pl
from jax.experimental.pallas import tpu as pltpu
from jax.experimental.pallas import tpu_sc as plsc

NUM_EMB = 1000000
DIM = 64
B_TOTAL = 16384 * 50
NW = 32
B_PER_W = B_TOTAL // NW
IDX_ROW = 128
N_IDX_ROWS = B_PER_W // IDX_ROW
ROWS_PER_CHUNK = 512
GATHERS_PER_CHUNK = ROWS_PER_CHUNK // IDX_ROW
N_CHUNKS = B_PER_W // ROWS_PER_CHUNK


def _body(ids_hbm, table_hbm, out_hbm, idx_v, rows_v, sem):
    nc = 2
    wid = lax.axis_index("s") * nc + lax.axis_index("c")
    pltpu.sync_copy(ids_hbm.at[wid], idx_v)
    base = wid * B_PER_W

    def step(g, carry):
        handles = []
        for j in range(GATHERS_PER_CHUNK):
            h = pltpu.async_copy(
                table_hbm.at[idx_v.at[g * GATHERS_PER_CHUNK + j]],
                rows_v.at[pl.ds(j * IDX_ROW, IDX_ROW)],
                sem,
            )
            handles.append(h)
        for h in handles:
            h.wait()
        pltpu.sync_copy(rows_v, out_hbm.at[pl.ds(base + g * ROWS_PER_CHUNK,
                                                 ROWS_PER_CHUNK)])
        return carry

    lax.fori_loop(0, N_CHUNKS, step, 0)


def kernel(token_ids, embedding_weights):
    ids = token_ids.reshape(NW, N_IDX_ROWS, IDX_ROW).astype(jnp.int32)
    mesh = plsc.VectorSubcoreMesh(core_axis_name="c", subcore_axis_name="s")
    k = functools.partial(
        pl.kernel,
        mesh=mesh,
        out_type=jax.ShapeDtypeStruct((B_TOTAL, DIM), jnp.float32),
        scratch_types=[
            pltpu.VMEM((N_IDX_ROWS, IDX_ROW), jnp.int32),
            pltpu.VMEM((ROWS_PER_CHUNK, DIM), jnp.float32),
            pltpu.SemaphoreType.DMA,
        ],
        compiler_params=pltpu.CompilerParams(use_tc_tiling_on_sc=False),
    )(_body)
    out = k(ids, embedding_weights)
    return out.reshape(token_ids.shape + (DIM,))


# speedup vs baseline: 1.8767x; 1.0243x over previous
import functools

import jax
import jax.numpy as jnp
from jax import lax
from jax.experimental import pallas as pl
from jax.experimental.pallas import tpu as pltpu
from jax.experimental.pallas import tpu_sc as plsc

NUM_EMB = 1000000
DIM = 64
B_TOTAL = 16384 * 50
NW = 32
B_PER_W = B_TOTAL // NW
IDX_ROW = 128
N_IDX_ROWS = B_PER_W // IDX_ROW
ROWS_PER_CHUNK = 512
GATHERS_PER_CHUNK = ROWS_PER_CHUNK // IDX_ROW
N_CHUNKS = B_PER_W // ROWS_PER_CHUNK


def _body(ids_hbm, table_hbm, out_hbm, idx_v, rows0, rows1, sem0, sem1):
    nc = 2
    wid = lax.axis_index("s") * nc + lax.axis_index("c")
    pltpu.sync_copy(ids_hbm.at[wid], idx_v)
    base = wid * B_PER_W

    def fire(c, buf, sem):
        for j in range(GATHERS_PER_CHUNK):
            pltpu.async_copy(
                table_hbm.at[idx_v.at[c * GATHERS_PER_CHUNK + j]],
                buf.at[pl.ds(j * IDX_ROW, IDX_ROW)],
                sem,
            )

    def drain(buf, sem):
        for j in range(GATHERS_PER_CHUNK):
            pltpu.make_async_copy(
                table_hbm.at[idx_v.at[j]],
                buf.at[pl.ds(j * IDX_ROW, IDX_ROW)],
                sem,
            ).wait()

    def write(c, buf):
        pltpu.sync_copy(buf, out_hbm.at[pl.ds(base + c * ROWS_PER_CHUNK,
                                              ROWS_PER_CHUNK)])

    fire(0, rows0, sem0)

    def step(i, carry):
        c0 = 2 * i
        fire(c0 + 1, rows1, sem1)
        drain(rows0, sem0)
        write(c0, rows0)

        @pl.when(i < N_CHUNKS // 2 - 1)
        def _():
            fire(c0 + 2, rows0, sem0)

        drain(rows1, sem1)
        write(c0 + 1, rows1)
        return carry

    lax.fori_loop(0, N_CHUNKS // 2, step, 0)


def kernel(token_ids, embedding_weights):
    ids = token_ids.reshape(NW, N_IDX_ROWS, IDX_ROW).astype(jnp.int32)
    mesh = plsc.VectorSubcoreMesh(core_axis_name="c", subcore_axis_name="s")
    k = functools.partial(
        pl.kernel,
        mesh=mesh,
        out_type=jax.ShapeDtypeStruct((B_TOTAL, DIM), jnp.float32),
        scratch_types=[
            pltpu.VMEM((N_IDX_ROWS, IDX_ROW), jnp.int32),
            pltpu.VMEM((ROWS_PER_CHUNK, DIM), jnp.float32),
            pltpu.VMEM((ROWS_PER_CHUNK, DIM), jnp.float32),
            pltpu.SemaphoreType.DMA,
            pltpu.SemaphoreType.DMA,
        ],
        compiler_params=pltpu.CompilerParams(use_tc_tiling_on_sc=False),
    )(_body)
    out = k(ids, embedding_weights)
    return out.reshape(token_ids.shape + (DIM,))


# speedup vs baseline: 1.8770x; 1.0002x over previous
import functools

import jax
import jax.numpy as jnp
from jax import lax
from jax.experimental import pallas as pl
from jax.experimental.pallas import tpu as pltpu
from jax.experimental.pallas import tpu_sc as plsc

NUM_EMB = 1000000
DIM = 64
B_TOTAL = 16384 * 50
NW = 32
B_PER_W = B_TOTAL // NW
IDX_ROW = 512
N_IDX_ROWS = B_PER_W // IDX_ROW
ROWS_PER_CHUNK = 512
GATHERS_PER_CHUNK = ROWS_PER_CHUNK // IDX_ROW
N_CHUNKS = B_PER_W // ROWS_PER_CHUNK


def _body(ids_hbm, table_hbm, out_hbm, idx_v, rows0, rows1, sem0, sem1):
    nc = 2
    wid = lax.axis_index("s") * nc + lax.axis_index("c")
    pltpu.sync_copy(ids_hbm.at[wid], idx_v)
    base = wid * B_PER_W

    def fire(c, buf, sem):
        for j in range(GATHERS_PER_CHUNK):
            pltpu.async_copy(
                table_hbm.at[idx_v.at[c * GATHERS_PER_CHUNK + j]],
                buf.at[pl.ds(j * IDX_ROW, IDX_ROW)],
                sem,
            )

    def drain(buf, sem):
        for j in range(GATHERS_PER_CHUNK):
            pltpu.make_async_copy(
                table_hbm.at[idx_v.at[j]],
                buf.at[pl.ds(j * IDX_ROW, IDX_ROW)],
                sem,
            ).wait()

    def write(c, buf):
        pltpu.sync_copy(buf, out_hbm.at[pl.ds(base + c * ROWS_PER_CHUNK,
                                              ROWS_PER_CHUNK)])

    fire(0, rows0, sem0)

    def step(i, carry):
        c0 = 2 * i
        fire(c0 + 1, rows1, sem1)
        drain(rows0, sem0)
        write(c0, rows0)

        @pl.when(i < N_CHUNKS // 2 - 1)
        def _():
            fire(c0 + 2, rows0, sem0)

        drain(rows1, sem1)
        write(c0 + 1, rows1)
        return carry

    lax.fori_loop(0, N_CHUNKS // 2, step, 0)


def kernel(token_ids, embedding_weights):
    ids = token_ids.reshape(NW, N_IDX_ROWS, IDX_ROW).astype(jnp.int32)
    mesh = plsc.VectorSubcoreMesh(core_axis_name="c", subcore_axis_name="s")
    k = functools.partial(
        pl.kernel,
        mesh=mesh,
        out_type=jax.ShapeDtypeStruct((B_TOTAL, DIM), jnp.float32),
        scratch_types=[
            pltpu.VMEM((N_IDX_ROWS, IDX_ROW), jnp.int32),
            pltpu.VMEM((ROWS_PER_CHUNK, DIM), jnp.float32),
            pltpu.VMEM((ROWS_PER_CHUNK, DIM), jnp.float32),
            pltpu.SemaphoreType.DMA,
            pltpu.SemaphoreType.DMA,
        ],
        compiler_params=pltpu.CompilerParams(use_tc_tiling_on_sc=False),
    )(_body)
    out = k(ids, embedding_weights)
    return out.reshape(token_ids.shape + (DIM,))
